# Initial kernel scaffold; baseline (speedup 1.0000x reference)
import jax
import jax.numpy as jnp
from jax.experimental import pallas as pl


def kernel(x_t, emission_logits):
    raise NotImplementedError("write your pallas kernel here")


# speedup vs baseline: 2.0327x; 2.0327x over previous
import functools

import jax
import jax.numpy as jnp
from jax import lax
from jax.experimental import pallas as pl
from jax.experimental.pallas import tpu as pltpu
from jax.experimental.pallas import tpu_sc as plsc

B = 16384
NC, NS, L = 2, 16, 16
NW = NC * NS
BPW = B // NW

_mesh = plsc.VectorSubcoreMesh(core_axis_name="c", subcore_axis_name="s")


@functools.partial(
    pl.kernel,
    mesh=_mesh,
    compiler_params=pltpu.CompilerParams(needs_layout_passes=False),
    out_type=jax.ShapeDtypeStruct((2 * B,), jnp.float32),
    scratch_types=[
        pltpu.VMEM((L,), jnp.float32),
        pltpu.VMEM((L,), jnp.float32),
        pltpu.VMEM((BPW,), jnp.int32),
        pltpu.VMEM((2 * BPW,), jnp.float32)
    ],
)
def _emission_sc(logits_hbm, x_hbm, out_hbm, tbl_v, tmp_v, x_v, o_v):
    wid = lax.axis_index("s") * NC + lax.axis_index("c")
    base = wid * BPW
    pltpu.sync_copy(x_hbm.at[pl.ds(base, BPW)], x_v)
    pltpu.sync_copy(logits_hbm, tbl_v)

    lanes = lax.iota(jnp.int32, L)
    partner = lanes ^ 1
    v = tbl_v[...]
    tmp_v[...] = v
    pv = plsc.load_gather(tmp_v, [partner])
    e = jnp.exp(v - jnp.maximum(v, pv))
    tmp_v[...] = e
    ep = plsc.load_gather(tmp_v, [partner])
    p = e / (e + ep)
    tmp_v[...] = p
    one = lanes * 0 + 1
    p00 = plsc.load_gather(tmp_v, [one + 3])
    p01 = plsc.load_gather(tmp_v, [one + 4])
    p10 = plsc.load_gather(tmp_v, [one + 5])
    p11 = plsc.load_gather(tmp_v, [one + 6])

    def body(i, carry):
        xv = plsc.load_gather(x_v, [lanes + i * L])
        msk = xv == 0
        o0 = jnp.where(msk, p00, p01)
        o1 = jnp.where(msk, p10, p11)
        idx0 = (lanes + i * L) * 2
        plsc.store_scatter(o_v, [idx0], o0)
        plsc.store_scatter(o_v, [idx0 + 1], o1)
        return carry

    lax.fori_loop(0, BPW // L, body, 0)
    pltpu.sync_copy(o_v, out_hbm.at[pl.ds(2 * base, 2 * BPW)])


def kernel(x_t, emission_logits):
    logits_flat = jnp.zeros((L,), jnp.float32).at[4:8].set(
        emission_logits.reshape(-1).astype(jnp.float32))
    out = _emission_sc(logits_flat, x_t.astype(jnp.int32))
    return out.reshape(B, 2)


# speedup vs baseline: 2.5521x; 1.2555x over previous
import functools

import jax
import jax.numpy as jnp
from jax import lax
from jax.experimental import pallas as pl
from jax.experimental.pallas import tpu as pltpu
from jax.experimental.pallas import tpu_sc as plsc

B = 16384
NC, NS, L = 2, 16, 16
NW = NC * NS
BPW = B // NW

_mesh = plsc.VectorSubcoreMesh(core_axis_name="c", subcore_axis_name="s")


@functools.partial(
    pl.kernel,
    mesh=_mesh,
    compiler_params=pltpu.CompilerParams(needs_layout_passes=False),
    out_type=jax.ShapeDtypeStruct((B, 2), jnp.float32),
    scratch_types=[
        pltpu.VMEM((L,), jnp.float32),
        pltpu.VMEM((L,), jnp.float32),
        pltpu.VMEM((BPW,), jnp.int32),
        pltpu.VMEM((BPW, 2), jnp.float32)
    ],
)
def _emission_sc(logits_hbm, x_hbm, out_hbm, tbl_v, tmp_v, x_v, o_v):
    wid = lax.axis_index("s") * NC + lax.axis_index("c")
    base = wid * BPW
    pltpu.sync_copy(x_hbm.at[pl.ds(base, BPW)], x_v)
    pltpu.sync_copy(logits_hbm, tbl_v)

    lanes = lax.iota(jnp.int32, L)
    partner = lanes ^ 1
    v = tbl_v[...]
    tmp_v[...] = v
    pv = plsc.load_gather(tmp_v, [partner])
    e = jnp.exp(v - jnp.maximum(v, pv))
    tmp_v[...] = e
    ep = plsc.load_gather(tmp_v, [partner])
    p = e / (e + ep)
    tmp_v[...] = p
    one = lanes * 0 + 1
    p00 = plsc.load_gather(tmp_v, [one + 3])
    p01 = plsc.load_gather(tmp_v, [one + 4])
    p10 = plsc.load_gather(tmp_v, [one + 5])
    p11 = plsc.load_gather(tmp_v, [one + 6])

    col0 = lax.shift_right_logical(plsc.load_gather(x_v, [lanes]), 31)
    col1 = col0 + 1

    def body(i, carry):
        rows = lanes + i * L
        xv = plsc.load_gather(x_v, [rows])
        msk = xv == 0
        o0 = jnp.where(msk, p00, p01)
        o1 = jnp.where(msk, p10, p11)
        plsc.store_scatter(o_v, [rows, col0], o0)
        plsc.store_scatter(o_v, [rows, col1], o1)
        return carry

    lax.fori_loop(0, BPW // L, body, 0)
    pltpu.sync_copy(o_v, out_hbm.at[pl.ds(base, BPW)])


def kernel(x_t, emission_logits):
    logits_flat = jnp.zeros((L,), jnp.float32).at[4:8].set(
        emission_logits.reshape(-1).astype(jnp.float32))
    return _emission_sc(logits_flat, x_t.astype(jnp.int32))


# speedup vs baseline: 3.6368x; 1.4250x over previous
import functools

import jax
import jax.numpy as jnp
from jax import lax
from jax.experimental import pallas as pl
from jax.experimental.pallas import tpu as pltpu
from jax.experimental.pallas import tpu_sc as plsc

B = 16384
NC, NS, L = 2, 16, 16
NW = NC * NS
BPW = B // NW

_mesh = plsc.VectorSubcoreMesh(core_axis_name="c", subcore_axis_name="s")


@functools.partial(
    pl.kernel,
    mesh=_mesh,
    compiler_params=pltpu.CompilerParams(needs_layout_passes=False),
    out_type=jax.ShapeDtypeStruct((2, B), jnp.float32),
    scratch_types=[
        pltpu.VMEM((L,), jnp.float32),
        pltpu.VMEM((L,), jnp.float32),
        pltpu.VMEM((BPW,), jnp.int32),
        pltpu.VMEM((BPW,), jnp.float32),
        pltpu.VMEM((BPW,), jnp.float32),
    ],
)
def _emission_sc(logits_hbm, x_hbm, out_hbm, tbl_v, tmp_v, x_v, a_v, b_v):
    wid = lax.axis_index("s") * NC + lax.axis_index("c")
    base = wid * BPW
    pltpu.sync_copy(x_hbm.at[pl.ds(base, BPW)], x_v)
    pltpu.sync_copy(logits_hbm, tbl_v)

    lanes = lax.iota(jnp.int32, L)
    partner = lanes ^ 1
    v = tbl_v[...]
    tmp_v[...] = v
    pv = plsc.load_gather(tmp_v, [partner])
    e = jnp.exp(v - jnp.maximum(v, pv))
    tmp_v[...] = e
    ep = plsc.load_gather(tmp_v, [partner])
    p = e / (e + ep)
    tmp_v[...] = p
    one = lanes * 0 + 1
    p00 = plsc.load_gather(tmp_v, [one + 3])
    p01 = plsc.load_gather(tmp_v, [one + 4])
    p10 = plsc.load_gather(tmp_v, [one + 5])
    p11 = plsc.load_gather(tmp_v, [one + 6])

    def body(i, carry):
        xv = plsc.load_gather(x_v, [lanes + i * L])
        msk = xv == 0
        a_v[pl.ds(i * L, L)] = jnp.where(msk, p00, p01)
        b_v[pl.ds(i * L, L)] = jnp.where(msk, p10, p11)
        return carry

    lax.fori_loop(0, BPW // L, body, 0)
    pltpu.sync_copy(a_v, out_hbm.at[0, pl.ds(base, BPW)])
    pltpu.sync_copy(b_v, out_hbm.at[1, pl.ds(base, BPW)])


def kernel(x_t, emission_logits):
    logits_flat = jnp.zeros((L,), jnp.float32).at[4:8].set(
        emission_logits.reshape(-1).astype(jnp.float32))
    return _emission_sc(logits_flat, x_t.astype(jnp.int32)).T


# speedup vs baseline: 3.7548x; 1.0325x over previous
import functools

import jax
import jax.numpy as jnp
from jax import lax
from jax.experimental import pallas as pl
from jax.experimental.pallas import tpu as pltpu
from jax.experimental.pallas import tpu_sc as plsc

B = 16384
NC, NS, L = 2, 16, 16
NW = NC * NS
BPW = B // NW

_mesh = plsc.VectorSubcoreMesh(core_axis_name="c", subcore_axis_name="s")


@functools.partial(
    pl.kernel,
    mesh=_mesh,
    compiler_params=pltpu.CompilerParams(needs_layout_passes=False),
    out_type=jax.ShapeDtypeStruct((2, B), jnp.float32),
    scratch_types=[
        pltpu.VMEM((L,), jnp.float32),
        pltpu.VMEM((L,), jnp.float32),
        pltpu.VMEM((BPW,), jnp.int32),
        pltpu.VMEM((BPW,), jnp.float32),
        pltpu.VMEM((BPW,), jnp.float32),
    ],
)
def _emission_sc(logits_hbm, x_hbm, out_hbm, tbl_v, tmp_v, x_v, a_v, b_v):
    wid = lax.axis_index("s") * NC + lax.axis_index("c")
    base = wid * BPW
    pltpu.sync_copy(x_hbm.at[pl.ds(base, BPW)], x_v)
    pltpu.sync_copy(logits_hbm, tbl_v.at[pl.ds(8, 4)])

    lanes = lax.iota(jnp.int32, L)
    partner = lanes ^ 1
    v = tbl_v[...]
    tmp_v[...] = v
    pv = plsc.load_gather(tmp_v, [partner])
    e = jnp.exp(v - jnp.maximum(v, pv))
    tmp_v[...] = e
    ep = plsc.load_gather(tmp_v, [partner])
    p = e / (e + ep)
    tmp_v[...] = p
    one = lanes * 0 + 1
    p00 = plsc.load_gather(tmp_v, [one + 7])
    p01 = plsc.load_gather(tmp_v, [one + 8])
    p10 = plsc.load_gather(tmp_v, [one + 9])
    p11 = plsc.load_gather(tmp_v, [one + 10])

    def body(i, carry):
        xv = plsc.load_gather(x_v, [lanes + i * L])
        msk = xv == 0
        a_v[pl.ds(i * L, L)] = jnp.where(msk, p00, p01)
        b_v[pl.ds(i * L, L)] = jnp.where(msk, p10, p11)
        return carry

    lax.fori_loop(0, BPW // L, body, 0)
    pltpu.sync_copy(a_v, out_hbm.at[0, pl.ds(base, BPW)])
    pltpu.sync_copy(b_v, out_hbm.at[1, pl.ds(base, BPW)])


def kernel(x_t, emission_logits):
    logits_flat = emission_logits.reshape(-1).astype(jnp.float32)
    return _emission_sc(logits_flat, x_t.astype(jnp.int32)).T


# speedup vs baseline: 3.9602x; 1.0547x over previous
import functools

import jax
import jax.numpy as jnp
from jax import lax
from jax.experimental import pallas as pl
from jax.experimental.pallas import tpu as pltpu
from jax.experimental.pallas import tpu_sc as plsc

B = 16384
NC, NS, L = 1, 16, 16
NW = NC * NS
BPW = B // NW

_mesh = plsc.VectorSubcoreMesh(core_axis_name="c", subcore_axis_name="s",
                               num_cores=1)


@functools.partial(
    pl.kernel,
    mesh=_mesh,
    compiler_params=pltpu.CompilerParams(needs_layout_passes=False),
    out_type=jax.ShapeDtypeStruct((2, B), jnp.float32),
    scratch_types=[
        pltpu.VMEM((L,), jnp.float32),
        pltpu.VMEM((L,), jnp.float32),
        pltpu.VMEM((BPW,), jnp.int32),
        pltpu.VMEM((BPW,), jnp.float32),
        pltpu.VMEM((BPW,), jnp.float32),
    ],
)
def _emission_sc(logits_hbm, x_hbm, out_hbm, tbl_v, tmp_v, x_v, a_v, b_v):
    wid = lax.axis_index("s") * NC + lax.axis_index("c")
    base = wid * BPW
    pltpu.sync_copy(x_hbm.at[pl.ds(base, BPW)], x_v)
    pltpu.sync_copy(logits_hbm, tbl_v.at[pl.ds(8, 4)])

    lanes = lax.iota(jnp.int32, L)
    partner = lanes ^ 1
    v = tbl_v[...]
    tmp_v[...] = v
    pv = plsc.load_gather(tmp_v, [partner])
    e = jnp.exp(v - jnp.maximum(v, pv))
    tmp_v[...] = e
    ep = plsc.load_gather(tmp_v, [partner])
    p = e / (e + ep)
    tmp_v[...] = p
    one = lanes * 0 + 1
    p00 = plsc.load_gather(tmp_v, [one + 7])
    p01 = plsc.load_gather(tmp_v, [one + 8])
    p10 = plsc.load_gather(tmp_v, [one + 9])
    p11 = plsc.load_gather(tmp_v, [one + 10])

    def body(i, carry):
        xv = plsc.load_gather(x_v, [lanes + i * L])
        msk = xv == 0
        a_v[pl.ds(i * L, L)] = jnp.where(msk, p00, p01)
        b_v[pl.ds(i * L, L)] = jnp.where(msk, p10, p11)
        return carry

    lax.fori_loop(0, BPW // L, body, 0)
    pltpu.sync_copy(a_v, out_hbm.at[0, pl.ds(base, BPW)])
    pltpu.sync_copy(b_v, out_hbm.at[1, pl.ds(base, BPW)])


def kernel(x_t, emission_logits):
    logits_flat = emission_logits.reshape(-1).astype(jnp.float32)
    return _emission_sc(logits_flat, x_t.astype(jnp.int32)).T


# speedup vs baseline: 3.9639x; 1.0010x over previous
import functools

import jax
import jax.numpy as jnp
from jax import lax
from jax.experimental import pallas as pl
from jax.experimental.pallas import tpu as pltpu
from jax.experimental.pallas import tpu_sc as plsc

B = 16384
NC, NS, L = 1, 16, 16
NW = NC * NS
BPW = B // NW

_mesh = plsc.VectorSubcoreMesh(core_axis_name="c", subcore_axis_name="s",
                               num_cores=1)


@functools.partial(
    pl.kernel,
    mesh=_mesh,
    compiler_params=pltpu.CompilerParams(needs_layout_passes=False),
    out_type=jax.ShapeDtypeStruct((2, B), jnp.float32),
    scratch_types=[
        pltpu.VMEM((2 * L,), jnp.float32),
        pltpu.VMEM((BPW,), jnp.int32),
        pltpu.VMEM((2 * BPW,), jnp.float32)
    ],
)
def _emission_sc(logits_hbm, x_hbm, out_hbm, t_v, x_v, o_v):
    wid = lax.axis_index("s") * NC + lax.axis_index("c")
    base = wid * BPW
    pltpu.sync_copy(x_hbm.at[pl.ds(base, BPW)], x_v)
    pltpu.sync_copy(logits_hbm, t_v.at[pl.ds(24, 4)])

    lanes = lax.iota(jnp.int32, L)
    partner = lanes ^ 1
    v = t_v[pl.ds(16, L)]
    t_v[pl.ds(0, L)] = v
    pv = plsc.load_gather(t_v, [partner])
    e = jnp.exp(v - jnp.maximum(v, pv))
    t_v[pl.ds(0, L)] = e
    ep = plsc.load_gather(t_v, [partner])
    p = e / (e + ep)
    t_v[pl.ds(0, L)] = p
    one = lanes * 0 + 1
    p00 = plsc.load_gather(t_v, [one + 7])
    p01 = plsc.load_gather(t_v, [one + 8])
    p10 = plsc.load_gather(t_v, [one + 9])
    p11 = plsc.load_gather(t_v, [one + 10])

    def body(i, carry):
        xv = plsc.load_gather(x_v, [lanes + i * L])
        msk = xv == 0
        o_v[pl.ds(i * L, L)] = jnp.where(msk, p00, p01)
        o_v[pl.ds(BPW + i * L, L)] = jnp.where(msk, p10, p11)
        return carry

    lax.fori_loop(0, BPW // L, body, 0)
    pltpu.sync_copy(o_v.at[pl.ds(0, BPW)], out_hbm.at[0, pl.ds(base, BPW)])
    pltpu.sync_copy(o_v.at[pl.ds(BPW, BPW)], out_hbm.at[1, pl.ds(base, BPW)])


def kernel(x_t, emission_logits):
    logits_flat = emission_logits.reshape(-1).astype(jnp.float32)
    return _emission_sc(logits_flat, x_t.astype(jnp.int32)).T
